# baseline (speedup 1.0000x reference)
import functools

import jax
import jax.numpy as jnp
from jax import lax
from jax.experimental import pallas as pl
from jax.experimental.pallas import tpu as pltpu
from jax.experimental.pallas import tpu_sc as plsc

N = 10000
D = 128
H = 128
G = 64
E = 320000

NC = 2
NS = 16
NW = NC * NS
CHUNK = 128
NCH = 80
NBUF = 2
HALF = NCH // 2
E_PAD = NW * NCH * CHUNK
PAD = E_PAD - E
ACC_ROWS = 10240
ZROWS = ACC_ROWS // NS
DROWS = 624
DTAIL = N - DROWS * NS
CW = 16

R = 200
NB = N // R


def _sc_mesh():
    return plsc.VectorSubcoreMesh(core_axis_name="c", subcore_axis_name="s")


def _deg_count(dst3, zo):
    @functools.partial(
        pl.kernel,
        out_type=jax.ShapeDtypeStruct((NC, N, CW), jnp.float32),
        mesh=_sc_mesh(),
        scratch_types=[
            pltpu.VMEM((NCH, CHUNK), jnp.int32),
            pltpu.VMEM((2, CHUNK, CW), jnp.float32),
            pltpu.VMEM_SHARED((ACC_ROWS, CW), jnp.float32),
        ],
    )
    def k(dst_hbm, zo_hbm, out_hbm, idx_v, zo_v, acc_s):
        c = lax.axis_index("c")
        s = lax.axis_index("s")
        wid = s * NC + c
        pltpu.sync_copy(zo_hbm, zo_v)
        pltpu.sync_copy(dst_hbm.at[wid], idx_v)
        base = s * ZROWS
        for t in range(ZROWS // CHUNK):
            pltpu.sync_copy(zo_v.at[0], acc_s.at[pl.ds(base + t * CHUNK, CHUNK)])
        rem = ZROWS % CHUNK
        if rem:
            pltpu.sync_copy(zo_v.at[0, pl.ds(0, rem)],
                            acc_s.at[pl.ds(base + ZROWS - rem, rem)])
        plsc.subcore_barrier()

        def body(i, carry):
            pltpu.sync_copy(zo_v.at[1], acc_s.at[idx_v.at[i]], add=True)
            return carry

        lax.fori_loop(0, NCH, body, 0)
        plsc.subcore_barrier()
        pltpu.sync_copy(acc_s.at[pl.ds(s * DROWS, DROWS)],
                        out_hbm.at[c, pl.ds(s * DROWS, DROWS)])

        @pl.when(s == 0)
        def _():
            pltpu.sync_copy(acc_s.at[pl.ds(DROWS * NS, DTAIL)],
                            out_hbm.at[c, pl.ds(DROWS * NS, DTAIL)])

    return k(dst3, zo)


def _edge_pass(p, src3, dst3, zeros_rows):
    @functools.partial(
        pl.kernel,
        out_type=jax.ShapeDtypeStruct((NC, N, H), jnp.float32),
        mesh=_sc_mesh(),
        scratch_types=[
            pltpu.VMEM((HALF, CHUNK), jnp.int32),
            pltpu.VMEM((HALF, CHUNK), jnp.int32),
            pltpu.VMEM((CHUNK, H), jnp.float32),
            pltpu.VMEM((CHUNK, H), jnp.float32),
            pltpu.VMEM_SHARED((ACC_ROWS, H), jnp.float32),
            pltpu.SemaphoreType.DMA,
            pltpu.SemaphoreType.DMA,
        ],
    )
    def k(p_hbm, src_hbm, dst_hbm, z_hbm, out_hbm,
          src_v, dst_v, buf0, buf1, acc_s, sem0, sem1):
        c = lax.axis_index("c")
        s = lax.axis_index("s")
        wid = s * NC + c
        bufs = (buf0, buf1)
        sems = (sem0, sem1)
        pltpu.sync_copy(src_hbm.at[wid, pl.ds(0, HALF)], src_v)
        for b in range(1, NBUF):
            pltpu.async_copy(p_hbm.at[src_v.at[b]], bufs[b], sems[b])
        pltpu.sync_copy(dst_hbm.at[wid, pl.ds(0, HALF)], dst_v)
        pltpu.sync_copy(z_hbm, buf0)
        base = s * ZROWS
        for t in range(ZROWS // CHUNK):
            pltpu.sync_copy(buf0, acc_s.at[pl.ds(base + t * CHUNK, CHUNK)])
        rem = ZROWS % CHUNK
        if rem:
            pltpu.sync_copy(buf0.at[pl.ds(0, rem)],
                            acc_s.at[pl.ds(base + ZROWS - rem, rem)])
        pltpu.async_copy(p_hbm.at[src_v.at[0]], buf0, sem0)
        plsc.subcore_barrier()

        def run_half():
            def body(it, carry):
                g = it * NBUF
                for b in range(NBUF):
                    i = g + b
                    pltpu.make_async_copy(p_hbm.at[src_v.at[i]], bufs[b],
                                          sems[b]).wait()
                    pltpu.sync_copy(bufs[b], acc_s.at[dst_v.at[i]], add=True)
                    nxt = i + NBUF

                    @pl.when(nxt < HALF)
                    def _():
                        pltpu.async_copy(p_hbm.at[src_v.at[nxt]],
                                         bufs[b], sems[b])
                return carry

            lax.fori_loop(0, HALF // NBUF, body, 0)

        run_half()
        pltpu.sync_copy(src_hbm.at[wid, pl.ds(HALF, HALF)], src_v)
        pltpu.sync_copy(dst_hbm.at[wid, pl.ds(HALF, HALF)], dst_v)
        for b in range(NBUF):
            pltpu.async_copy(p_hbm.at[src_v.at[b]], bufs[b], sems[b])
        run_half()
        plsc.subcore_barrier()
        pltpu.sync_copy(acc_s.at[pl.ds(s * DROWS, DROWS)],
                        out_hbm.at[c, pl.ds(s * DROWS, DROWS)])

        @pl.when(s == 0)
        def _():
            pltpu.sync_copy(acc_s.at[pl.ds(DROWS * NS, DTAIL)],
                            out_hbm.at[c, pl.ds(DROWS * NS, DTAIL)])

    return k(p, src3, dst3, zeros_rows)


def _dinv(cA, cB):
    return lax.rsqrt(cA[:, 0:1] + cB[:, 0:1] + 1.0)


def _matmul1_body(x_ref, w_ref, q_ref):
    q_ref[...] = jnp.dot(x_ref[...], w_ref[...],
                         preferred_element_type=jnp.float32)


def _matmul1(x, W1):
    return pl.pallas_call(
        _matmul1_body,
        grid=(NB,),
        in_specs=[
            pl.BlockSpec((R, D), lambda i: (i, 0)),
            pl.BlockSpec((D, H), lambda i: (0, 0)),
        ],
        out_specs=pl.BlockSpec((R, H), lambda i: (i, 0)),
        out_shape=jax.ShapeDtypeStruct((N, H), jnp.float32),
    )(x, W1)


def _scale_body(cA_ref, cB_ref, q_ref, p_ref):
    p_ref[...] = q_ref[...] * _dinv(cA_ref[...], cB_ref[...])


def _scale(cA, cB, q):
    return pl.pallas_call(
        _scale_body,
        grid=(NB,),
        in_specs=[
            pl.BlockSpec((R, CW), lambda i: (i, 0)),
            pl.BlockSpec((R, CW), lambda i: (i, 0)),
            pl.BlockSpec((R, H), lambda i: (i, 0)),
        ],
        out_specs=pl.BlockSpec((R, H), lambda i: (i, 0)),
        out_shape=jax.ShapeDtypeStruct((N, H), jnp.float32),
    )(cA, cB, q)


def _combine_body(cA_ref, cB_ref, aA_ref, aB_ref, p_ref, b_ref, w_ref, o_ref):
    d = _dinv(cA_ref[...], cB_ref[...])
    h = jax.nn.relu(d * (aA_ref[...] + aB_ref[...] + p_ref[...]) + b_ref[...])
    o_ref[...] = jnp.dot(h, w_ref[...], preferred_element_type=jnp.float32) * d


def _combine_prep(cA, cB, aA, aB, p, b1, W2):
    return pl.pallas_call(
        _combine_body,
        grid=(NB,),
        in_specs=[
            pl.BlockSpec((R, CW), lambda i: (i, 0)),
            pl.BlockSpec((R, CW), lambda i: (i, 0)),
            pl.BlockSpec((R, H), lambda i: (i, 0)),
            pl.BlockSpec((R, H), lambda i: (i, 0)),
            pl.BlockSpec((R, H), lambda i: (i, 0)),
            pl.BlockSpec((1, H), lambda i: (0, 0)),
            pl.BlockSpec((H, H), lambda i: (0, 0)),
        ],
        out_specs=pl.BlockSpec((R, H), lambda i: (i, 0)),
        out_shape=jax.ShapeDtypeStruct((N, H), jnp.float32),
    )(cA, cB, aA, aB, p, b1, W2)


def _final_body(cA_ref, cB_ref, aA_ref, aB_ref, p_ref, b_ref, bat_ref,
                wt_ref, bo_ref, o_ref, pool_acc, cnt_acc):
    i = pl.program_id(0)
    d = _dinv(cA_ref[...], cB_ref[...])
    h = jax.nn.relu(d * (aA_ref[...] + aB_ref[...] + p_ref[...]) + b_ref[...])
    bat = bat_ref[0, 0, :]
    oh = (bat[:, None] == lax.broadcasted_iota(jnp.int32, (R, G), 1)
          ).astype(jnp.float32)
    pool = lax.dot_general(oh, h, (((0,), (0,)), ((), ())),
                           preferred_element_type=jnp.float32)
    cnt = jnp.broadcast_to(jnp.sum(oh, axis=0)[:, None], (G, H))

    @pl.when(i == 0)
    def _():
        pool_acc[...] = pool
        cnt_acc[...] = cnt

    @pl.when(i > 0)
    def _():
        pool_acc[...] += pool
        cnt_acc[...] += cnt

    @pl.when(i == NB - 1)
    def _():
        pooled = pool_acc[...] / jnp.maximum(cnt_acc[...], 1.0)
        z = jnp.sum(pooled * wt_ref[...], axis=1, keepdims=True) + bo_ref[...]
        o_ref[...] = 1.0 / (1.0 + jnp.exp(-z))


def _final(cA, cB, aA, aB, p, b2, bat3, WoutT, bout):
    return pl.pallas_call(
        _final_body,
        grid=(NB,),
        in_specs=[
            pl.BlockSpec((R, CW), lambda i: (i, 0)),
            pl.BlockSpec((R, CW), lambda i: (i, 0)),
            pl.BlockSpec((R, H), lambda i: (i, 0)),
            pl.BlockSpec((R, H), lambda i: (i, 0)),
            pl.BlockSpec((R, H), lambda i: (i, 0)),
            pl.BlockSpec((1, H), lambda i: (0, 0)),
            pl.BlockSpec((1, 1, R), lambda i: (i, 0, 0)),
            pl.BlockSpec((1, H), lambda i: (0, 0)),
            pl.BlockSpec((1, 1), lambda i: (0, 0)),
        ],
        out_specs=pl.BlockSpec((G, 1), lambda i: (0, 0)),
        out_shape=jax.ShapeDtypeStruct((G, 1), jnp.float32),
        scratch_shapes=[
            pltpu.VMEM((G, H), jnp.float32),
            pltpu.VMEM((G, H), jnp.float32),
        ],
    )(cA, cB, aA, aB, p, b2, bat3, WoutT, bout)


def kernel(x, edge_index, batch, W1, b1, W2, b2, W_out, b_out):
    pad_ids = jnp.arange(PAD, dtype=jnp.int32)
    src_pad = pad_ids % jnp.int32(N)
    dst_pad = jnp.int32(N) + (pad_ids % jnp.int32(NS))
    src3 = jnp.concatenate([edge_index[0], src_pad]).reshape(NW, NCH, CHUNK)
    dst3 = jnp.concatenate([edge_index[1], dst_pad]).reshape(NW, NCH, CHUNK)

    zo = jnp.stack([jnp.zeros((CHUNK, CW), jnp.float32),
                    jnp.ones((CHUNK, CW), jnp.float32)])
    zeros_rows = jnp.zeros((CHUNK, H), jnp.float32)

    q1 = _matmul1(x, W1)
    cnt = _deg_count(dst3, zo)
    cA, cB = cnt[0], cnt[1]

    p1 = _scale(cA, cB, q1)
    acc1 = _edge_pass(p1, src3, dst3, zeros_rows)
    p2 = _combine_prep(cA, cB, acc1[0], acc1[1], p1, b1.reshape(1, H), W2)
    acc2 = _edge_pass(p2, src3, dst3, zeros_rows)

    bat3 = batch.reshape(NB, 1, R)
    out = _final(cA, cB, acc2[0], acc2[1], p2, b2.reshape(1, H),
                 bat3, W_out.reshape(1, H), b_out.reshape(1, 1))
    return out


# speedup vs baseline: 1.0615x; 1.0615x over previous
import functools

import jax
import jax.numpy as jnp
from jax import lax
from jax.experimental import pallas as pl
from jax.experimental.pallas import tpu as pltpu
from jax.experimental.pallas import tpu_sc as plsc

N = 10000
D = 128
H = 128
G = 64
E = 320000

NC = 2
NS = 16
NW = NC * NS
CHUNK = 128
NCH = 80
NBUF = 2
HALF = NCH // 2
E_PAD = NW * NCH * CHUNK
PAD = E_PAD - E
ACC_ROWS = 10240
ZROWS = ACC_ROWS // NS
DROWS = 624
DTAIL = N - DROWS * NS
CW = 16

R = 200
NB = N // R


def _sc_mesh():
    return plsc.VectorSubcoreMesh(core_axis_name="c", subcore_axis_name="s")


def _deg_count(dst3, zo):
    @functools.partial(
        pl.kernel,
        out_type=jax.ShapeDtypeStruct((NC, N, CW), jnp.float32),
        mesh=_sc_mesh(),
        scratch_types=[
            pltpu.VMEM((NCH, CHUNK), jnp.int32),
            pltpu.VMEM((2, CHUNK, CW), jnp.float32),
            pltpu.VMEM_SHARED((ACC_ROWS, CW), jnp.float32),
        ],
    )
    def k(dst_hbm, zo_hbm, out_hbm, idx_v, zo_v, acc_s):
        c = lax.axis_index("c")
        s = lax.axis_index("s")
        wid = s * NC + c
        pltpu.sync_copy(zo_hbm, zo_v)
        pltpu.sync_copy(dst_hbm.at[wid], idx_v)
        base = s * ZROWS
        for t in range(ZROWS // CHUNK):
            pltpu.sync_copy(zo_v.at[0], acc_s.at[pl.ds(base + t * CHUNK, CHUNK)])
        rem = ZROWS % CHUNK
        if rem:
            pltpu.sync_copy(zo_v.at[0, pl.ds(0, rem)],
                            acc_s.at[pl.ds(base + ZROWS - rem, rem)])
        plsc.subcore_barrier()

        def body(i, carry):
            pltpu.sync_copy(zo_v.at[1], acc_s.at[idx_v.at[i]], add=True)
            return carry

        lax.fori_loop(0, NCH, body, 0)
        plsc.subcore_barrier()
        pltpu.sync_copy(acc_s.at[pl.ds(s * DROWS, DROWS)],
                        out_hbm.at[c, pl.ds(s * DROWS, DROWS)])

        @pl.when(s == 0)
        def _():
            pltpu.sync_copy(acc_s.at[pl.ds(DROWS * NS, DTAIL)],
                            out_hbm.at[c, pl.ds(DROWS * NS, DTAIL)])

    return k(dst3, zo)


def _edge_pass(p, src3, dst3, zeros_rows):
    @functools.partial(
        pl.kernel,
        out_type=jax.ShapeDtypeStruct((NC, N, H), jnp.float32),
        mesh=_sc_mesh(),
        scratch_types=[
            pltpu.VMEM((HALF, CHUNK), jnp.int32),
            pltpu.VMEM((HALF, CHUNK), jnp.int32),
            pltpu.VMEM((CHUNK, H), jnp.float32),
            pltpu.VMEM((CHUNK, H), jnp.float32),
            pltpu.VMEM_SHARED((ACC_ROWS, H), jnp.float32),
            pltpu.SemaphoreType.DMA,
            pltpu.SemaphoreType.DMA,
        ],
    )
    def k(p_hbm, src_hbm, dst_hbm, z_hbm, out_hbm,
          src_v, dst_v, buf0, buf1, acc_s, sem0, sem1):
        c = lax.axis_index("c")
        s = lax.axis_index("s")
        wid = s * NC + c
        bufs = (buf0, buf1)
        sems = (sem0, sem1)
        pltpu.sync_copy(src_hbm.at[wid, pl.ds(0, HALF)], src_v)
        for b in range(1, NBUF):
            pltpu.async_copy(p_hbm.at[src_v.at[b]], bufs[b], sems[b])
        pltpu.sync_copy(dst_hbm.at[wid, pl.ds(0, HALF)], dst_v)
        pltpu.sync_copy(z_hbm, buf0)
        base = s * ZROWS
        for t in range(ZROWS // CHUNK):
            pltpu.sync_copy(buf0, acc_s.at[pl.ds(base + t * CHUNK, CHUNK)])
        rem = ZROWS % CHUNK
        if rem:
            pltpu.sync_copy(buf0.at[pl.ds(0, rem)],
                            acc_s.at[pl.ds(base + ZROWS - rem, rem)])
        pltpu.async_copy(p_hbm.at[src_v.at[0]], buf0, sem0)
        plsc.subcore_barrier()

        def run_half():
            def body(it, carry):
                g = it * NBUF
                for b in range(NBUF):
                    i = g + b
                    pltpu.make_async_copy(p_hbm.at[src_v.at[i]], bufs[b],
                                          sems[b]).wait()
                    pltpu.sync_copy(bufs[b], acc_s.at[dst_v.at[i]], add=True)
                    nxt = i + NBUF

                    @pl.when(nxt < HALF)
                    def _():
                        pltpu.async_copy(p_hbm.at[src_v.at[nxt]],
                                         bufs[b], sems[b])
                return carry

            lax.fori_loop(0, HALF // NBUF, body, 0)

        run_half()
        pltpu.sync_copy(src_hbm.at[wid, pl.ds(HALF, HALF)], src_v)
        pltpu.sync_copy(dst_hbm.at[wid, pl.ds(HALF, HALF)], dst_v)
        for b in range(NBUF):
            pltpu.async_copy(p_hbm.at[src_v.at[b]], bufs[b], sems[b])
        run_half()
        plsc.subcore_barrier()
        pltpu.sync_copy(acc_s.at[pl.ds(s * DROWS, DROWS)],
                        out_hbm.at[c, pl.ds(s * DROWS, DROWS)])

        @pl.when(s == 0)
        def _():
            pltpu.sync_copy(acc_s.at[pl.ds(DROWS * NS, DTAIL)],
                            out_hbm.at[c, pl.ds(DROWS * NS, DTAIL)])

    return k(p, src3, dst3, zeros_rows)


def _dinv(cnt):
    return lax.rsqrt(cnt[0, :, 0:1] + cnt[1, :, 0:1] + 1.0)


def _prep1_body(cnt_ref, x_ref, w_ref, p_ref):
    d = _dinv(cnt_ref[...])
    p_ref[...] = jnp.dot(x_ref[...], w_ref[...],
                         preferred_element_type=jnp.float32) * d


def _prep1(cnt, x, W1):
    return pl.pallas_call(
        _prep1_body,
        grid=(NB,),
        in_specs=[
            pl.BlockSpec((NC, R, CW), lambda i: (0, i, 0)),
            pl.BlockSpec((R, D), lambda i: (i, 0)),
            pl.BlockSpec((D, H), lambda i: (0, 0)),
        ],
        out_specs=pl.BlockSpec((R, H), lambda i: (i, 0)),
        out_shape=jax.ShapeDtypeStruct((N, H), jnp.float32),
    )(cnt, x, W1)


def _combine_body(cnt_ref, acc_ref, p_ref, b_ref, w_ref, o_ref):
    d = _dinv(cnt_ref[...])
    h = jax.nn.relu(d * (acc_ref[0] + acc_ref[1] + p_ref[...]) + b_ref[...])
    o_ref[...] = jnp.dot(h, w_ref[...], preferred_element_type=jnp.float32) * d


def _combine_prep(cnt, acc, p, b1, W2):
    return pl.pallas_call(
        _combine_body,
        grid=(NB,),
        in_specs=[
            pl.BlockSpec((NC, R, CW), lambda i: (0, i, 0)),
            pl.BlockSpec((NC, R, H), lambda i: (0, i, 0)),
            pl.BlockSpec((R, H), lambda i: (i, 0)),
            pl.BlockSpec((1, H), lambda i: (0, 0)),
            pl.BlockSpec((H, H), lambda i: (0, 0)),
        ],
        out_specs=pl.BlockSpec((R, H), lambda i: (i, 0)),
        out_shape=jax.ShapeDtypeStruct((N, H), jnp.float32),
    )(cnt, acc, p, b1, W2)


def _final_body(cnt_ref, acc_ref, p_ref, b_ref, bat_ref,
                wt_ref, bo_ref, o_ref, pool_acc, cnt_acc):
    i = pl.program_id(0)
    d = _dinv(cnt_ref[...])
    h = jax.nn.relu(d * (acc_ref[0] + acc_ref[1] + p_ref[...]) + b_ref[...])
    bat = bat_ref[0, 0, :]
    oh = (bat[:, None] == lax.broadcasted_iota(jnp.int32, (R, G), 1)
          ).astype(jnp.float32)
    pool = lax.dot_general(oh, h, (((0,), (0,)), ((), ())),
                           preferred_element_type=jnp.float32)
    cnt = jnp.broadcast_to(jnp.sum(oh, axis=0)[:, None], (G, H))

    @pl.when(i == 0)
    def _():
        pool_acc[...] = pool
        cnt_acc[...] = cnt

    @pl.when(i > 0)
    def _():
        pool_acc[...] += pool
        cnt_acc[...] += cnt

    @pl.when(i == NB - 1)
    def _():
        pooled = pool_acc[...] / jnp.maximum(cnt_acc[...], 1.0)
        z = jnp.sum(pooled * wt_ref[...], axis=1, keepdims=True) + bo_ref[...]
        o_ref[...] = 1.0 / (1.0 + jnp.exp(-z))


def _final(cnt, acc, p, b2, bat3, WoutT, bout):
    return pl.pallas_call(
        _final_body,
        grid=(NB,),
        in_specs=[
            pl.BlockSpec((NC, R, CW), lambda i: (0, i, 0)),
            pl.BlockSpec((NC, R, H), lambda i: (0, i, 0)),
            pl.BlockSpec((R, H), lambda i: (i, 0)),
            pl.BlockSpec((1, H), lambda i: (0, 0)),
            pl.BlockSpec((1, 1, R), lambda i: (i, 0, 0)),
            pl.BlockSpec((1, H), lambda i: (0, 0)),
            pl.BlockSpec((1, 1), lambda i: (0, 0)),
        ],
        out_specs=pl.BlockSpec((G, 1), lambda i: (0, 0)),
        out_shape=jax.ShapeDtypeStruct((G, 1), jnp.float32),
        scratch_shapes=[
            pltpu.VMEM((G, H), jnp.float32),
            pltpu.VMEM((G, H), jnp.float32),
        ],
    )(cnt, acc, p, b2, bat3, WoutT, bout)


def kernel(x, edge_index, batch, W1, b1, W2, b2, W_out, b_out):
    pad_ids = jnp.arange(PAD, dtype=jnp.int32)
    src_pad = pad_ids % jnp.int32(N)
    dst_pad = jnp.int32(N) + (pad_ids % jnp.int32(NS))
    src3 = jnp.concatenate([edge_index[0], src_pad]).reshape(NW, NCH, CHUNK)
    dst3 = jnp.concatenate([edge_index[1], dst_pad]).reshape(NW, NCH, CHUNK)

    zo = jnp.stack([jnp.zeros((CHUNK, CW), jnp.float32),
                    jnp.ones((CHUNK, CW), jnp.float32)])
    zeros_rows = jnp.zeros((CHUNK, H), jnp.float32)

    cnt = _deg_count(dst3, zo)

    p1 = _prep1(cnt, x, W1)
    acc1 = _edge_pass(p1, src3, dst3, zeros_rows)
    p2 = _combine_prep(cnt, acc1, p1, b1.reshape(1, H), W2)
    acc2 = _edge_pass(p2, src3, dst3, zeros_rows)

    bat3 = batch.reshape(NB, 1, R)
    out = _final(cnt, acc2, p2, b2.reshape(1, H),
                 bat3, W_out.reshape(1, H), b_out.reshape(1, 1))
    return out


# speedup vs baseline: 1.1857x; 1.1170x over previous
import functools

import jax
import jax.numpy as jnp
from jax import lax
from jax.experimental import pallas as pl
from jax.experimental.pallas import tpu as pltpu
from jax.experimental.pallas import tpu_sc as plsc

N = 10000
D = 128
H = 128
G = 64
E = 320000

NC = 2
NS = 16
NW = NC * NS
CHUNK = 128
NCH = 80
NBUF = 2
HALF = NCH // 2
E_PAD = NW * NCH * CHUNK
PAD = E_PAD - E
ACC_ROWS = 10240
ZROWS = ACC_ROWS // NS
DROWS = 624
DTAIL = N - DROWS * NS
CW = 16

R = 400
NB = N // R


def _sc_mesh():
    return plsc.VectorSubcoreMesh(core_axis_name="c", subcore_axis_name="s")


def _deg_count(dst3, zo):
    @functools.partial(
        pl.kernel,
        out_type=jax.ShapeDtypeStruct((NC, N, CW), jnp.float32),
        mesh=_sc_mesh(),
        scratch_types=[
            pltpu.VMEM((NCH, CHUNK), jnp.int32),
            pltpu.VMEM((2, CHUNK, CW), jnp.float32),
            pltpu.VMEM_SHARED((ACC_ROWS, CW), jnp.float32),
        ],
    )
    def k(dst_hbm, zo_hbm, out_hbm, idx_v, zo_v, acc_s):
        c = lax.axis_index("c")
        s = lax.axis_index("s")
        wid = s * NC + c
        pltpu.sync_copy(zo_hbm, zo_v)
        pltpu.sync_copy(dst_hbm.at[wid], idx_v)
        base = s * ZROWS
        for t in range(ZROWS // CHUNK):
            pltpu.sync_copy(zo_v.at[0], acc_s.at[pl.ds(base + t * CHUNK, CHUNK)])
        rem = ZROWS % CHUNK
        if rem:
            pltpu.sync_copy(zo_v.at[0, pl.ds(0, rem)],
                            acc_s.at[pl.ds(base + ZROWS - rem, rem)])
        plsc.subcore_barrier()

        def body(i, carry):
            pltpu.sync_copy(zo_v.at[1], acc_s.at[idx_v.at[i]], add=True)
            return carry

        lax.fori_loop(0, NCH, body, 0)
        plsc.subcore_barrier()
        pltpu.sync_copy(acc_s.at[pl.ds(s * DROWS, DROWS)],
                        out_hbm.at[c, pl.ds(s * DROWS, DROWS)])

        @pl.when(s == 0)
        def _():
            pltpu.sync_copy(acc_s.at[pl.ds(DROWS * NS, DTAIL)],
                            out_hbm.at[c, pl.ds(DROWS * NS, DTAIL)])

    return k(dst3, zo)


def _edge_pass(p, src3, dst3, zeros_rows):
    @functools.partial(
        pl.kernel,
        out_type=jax.ShapeDtypeStruct((NC, N, H), jnp.float32),
        mesh=_sc_mesh(),
        scratch_types=[
            pltpu.VMEM((HALF, CHUNK), jnp.int32),
            pltpu.VMEM((HALF, CHUNK), jnp.int32),
            pltpu.VMEM((CHUNK, H), jnp.float32),
            pltpu.VMEM((CHUNK, H), jnp.float32),
            pltpu.VMEM_SHARED((ACC_ROWS, H), jnp.float32),
            pltpu.SemaphoreType.DMA,
            pltpu.SemaphoreType.DMA,
        ],
    )
    def k(p_hbm, src_hbm, dst_hbm, z_hbm, out_hbm,
          src_v, dst_v, buf0, buf1, acc_s, sem0, sem1):
        c = lax.axis_index("c")
        s = lax.axis_index("s")
        wid = s * NC + c
        bufs = (buf0, buf1)
        sems = (sem0, sem1)
        pltpu.sync_copy(src_hbm.at[wid, pl.ds(0, HALF)], src_v)
        for b in range(1, NBUF):
            pltpu.async_copy(p_hbm.at[src_v.at[b]], bufs[b], sems[b])
        pltpu.sync_copy(dst_hbm.at[wid, pl.ds(0, HALF)], dst_v)
        pltpu.sync_copy(z_hbm, buf0)
        base = s * ZROWS
        for t in range(ZROWS // CHUNK):
            pltpu.sync_copy(buf0, acc_s.at[pl.ds(base + t * CHUNK, CHUNK)])
        rem = ZROWS % CHUNK
        if rem:
            pltpu.sync_copy(buf0.at[pl.ds(0, rem)],
                            acc_s.at[pl.ds(base + ZROWS - rem, rem)])
        pltpu.async_copy(p_hbm.at[src_v.at[0]], buf0, sem0)
        plsc.subcore_barrier()

        def run_half():
            def body(it, carry):
                g = it * NBUF
                for b in range(NBUF):
                    i = g + b
                    pltpu.make_async_copy(p_hbm.at[src_v.at[i]], bufs[b],
                                          sems[b]).wait()
                    pltpu.sync_copy(bufs[b], acc_s.at[dst_v.at[i]], add=True)
                    nxt = i + NBUF

                    @pl.when(nxt < HALF)
                    def _():
                        pltpu.async_copy(p_hbm.at[src_v.at[nxt]],
                                         bufs[b], sems[b])
                return carry

            lax.fori_loop(0, HALF // NBUF, body, 0)

        run_half()
        pltpu.sync_copy(src_hbm.at[wid, pl.ds(HALF, HALF)], src_v)
        pltpu.sync_copy(dst_hbm.at[wid, pl.ds(HALF, HALF)], dst_v)
        for b in range(NBUF):
            pltpu.async_copy(p_hbm.at[src_v.at[b]], bufs[b], sems[b])
        run_half()
        plsc.subcore_barrier()
        pltpu.sync_copy(acc_s.at[pl.ds(s * DROWS, DROWS)],
                        out_hbm.at[c, pl.ds(s * DROWS, DROWS)])

        @pl.when(s == 0)
        def _():
            pltpu.sync_copy(acc_s.at[pl.ds(DROWS * NS, DTAIL)],
                            out_hbm.at[c, pl.ds(DROWS * NS, DTAIL)])

    return k(p, src3, dst3, zeros_rows)


def _dinv(cnt):
    return lax.rsqrt(cnt[0, :, 0:1] + cnt[1, :, 0:1] + 1.0)


def _prep1_body(cnt_ref, x_ref, w_ref, p_ref):
    d = _dinv(cnt_ref[...])
    p_ref[...] = jnp.dot(x_ref[...], w_ref[...],
                         preferred_element_type=jnp.float32) * d


def _prep1(cnt, x, W1):
    return pl.pallas_call(
        _prep1_body,
        grid=(NB,),
        in_specs=[
            pl.BlockSpec((NC, R, CW), lambda i: (0, i, 0)),
            pl.BlockSpec((R, D), lambda i: (i, 0)),
            pl.BlockSpec((D, H), lambda i: (0, 0)),
        ],
        out_specs=pl.BlockSpec((R, H), lambda i: (i, 0)),
        out_shape=jax.ShapeDtypeStruct((N, H), jnp.float32),
    )(cnt, x, W1)


def _combine_body(cnt_ref, acc_ref, p_ref, b_ref, w_ref, o_ref):
    d = _dinv(cnt_ref[...])
    h = jax.nn.relu(d * (acc_ref[0] + acc_ref[1] + p_ref[...]) + b_ref[...])
    o_ref[...] = jnp.dot(h, w_ref[...], preferred_element_type=jnp.float32) * d


def _combine_prep(cnt, acc, p, b1, W2):
    return pl.pallas_call(
        _combine_body,
        grid=(NB,),
        in_specs=[
            pl.BlockSpec((NC, R, CW), lambda i: (0, i, 0)),
            pl.BlockSpec((NC, R, H), lambda i: (0, i, 0)),
            pl.BlockSpec((R, H), lambda i: (i, 0)),
            pl.BlockSpec((1, H), lambda i: (0, 0)),
            pl.BlockSpec((H, H), lambda i: (0, 0)),
        ],
        out_specs=pl.BlockSpec((R, H), lambda i: (i, 0)),
        out_shape=jax.ShapeDtypeStruct((N, H), jnp.float32),
    )(cnt, acc, p, b1, W2)


def _final_body(cnt_ref, acc_ref, p_ref, b_ref, bat_ref,
                wt_ref, bo_ref, o_ref, pool_acc, cnt_acc):
    i = pl.program_id(0)
    d = _dinv(cnt_ref[...])
    h = jax.nn.relu(d * (acc_ref[0] + acc_ref[1] + p_ref[...]) + b_ref[...])
    bat = bat_ref[0, 0, :]
    oh = (bat[:, None] == lax.broadcasted_iota(jnp.int32, (R, G), 1)
          ).astype(jnp.float32)
    pool = lax.dot_general(oh, h, (((0,), (0,)), ((), ())),
                           preferred_element_type=jnp.float32)
    cnt = jnp.broadcast_to(jnp.sum(oh, axis=0)[:, None], (G, H))

    @pl.when(i == 0)
    def _():
        pool_acc[...] = pool
        cnt_acc[...] = cnt

    @pl.when(i > 0)
    def _():
        pool_acc[...] += pool
        cnt_acc[...] += cnt

    @pl.when(i == NB - 1)
    def _():
        pooled = pool_acc[...] / jnp.maximum(cnt_acc[...], 1.0)
        z = jnp.sum(pooled * wt_ref[...], axis=1, keepdims=True) + bo_ref[...]
        o_ref[...] = 1.0 / (1.0 + jnp.exp(-z))


def _final(cnt, acc, p, b2, bat3, WoutT, bout):
    return pl.pallas_call(
        _final_body,
        grid=(NB,),
        in_specs=[
            pl.BlockSpec((NC, R, CW), lambda i: (0, i, 0)),
            pl.BlockSpec((NC, R, H), lambda i: (0, i, 0)),
            pl.BlockSpec((R, H), lambda i: (i, 0)),
            pl.BlockSpec((1, H), lambda i: (0, 0)),
            pl.BlockSpec((1, 1, R), lambda i: (i, 0, 0)),
            pl.BlockSpec((1, H), lambda i: (0, 0)),
            pl.BlockSpec((1, 1), lambda i: (0, 0)),
        ],
        out_specs=pl.BlockSpec((G, 1), lambda i: (0, 0)),
        out_shape=jax.ShapeDtypeStruct((G, 1), jnp.float32),
        scratch_shapes=[
            pltpu.VMEM((G, H), jnp.float32),
            pltpu.VMEM((G, H), jnp.float32),
        ],
    )(cnt, acc, p, b2, bat3, WoutT, bout)


def kernel(x, edge_index, batch, W1, b1, W2, b2, W_out, b_out):
    pad_ids = jnp.arange(PAD, dtype=jnp.int32)
    src_pad = pad_ids % jnp.int32(N)
    dst_pad = jnp.int32(N) + (pad_ids % jnp.int32(NS))
    src3 = jnp.concatenate([edge_index[0], src_pad]).reshape(NW, NCH, CHUNK)
    dst3 = jnp.concatenate([edge_index[1], dst_pad]).reshape(NW, NCH, CHUNK)

    zo = jnp.stack([jnp.zeros((CHUNK, CW), jnp.float32),
                    jnp.ones((CHUNK, CW), jnp.float32)])
    zeros_rows = jnp.zeros((CHUNK, H), jnp.float32)

    cnt = _deg_count(dst3, zo)

    p1 = _prep1(cnt, x, W1)
    acc1 = _edge_pass(p1, src3, dst3, zeros_rows)
    p2 = _combine_prep(cnt, acc1, p1, b1.reshape(1, H), W2)
    acc2 = _edge_pass(p2, src3, dst3, zeros_rows)

    bat3 = batch.reshape(NB, 1, R)
    out = _final(cnt, acc2, p2, b2.reshape(1, H),
                 bat3, W_out.reshape(1, H), b_out.reshape(1, 1))
    return out


# speedup vs baseline: 1.2773x; 1.0773x over previous
import functools

import jax
import jax.numpy as jnp
from jax import lax
from jax.experimental import pallas as pl
from jax.experimental.pallas import tpu as pltpu
from jax.experimental.pallas import tpu_sc as plsc

N = 10000
D = 128
H = 128
G = 64
E = 320000

NC = 2
NS = 16
NW = NC * NS
CHUNK = 128
NCH = 80
NBUF = 2
HALF = NCH // 2
E_PAD = NW * NCH * CHUNK
PAD = E_PAD - E
ACC_ROWS = 10240
ZROWS = ACC_ROWS // NS
DROWS = 624
DTAIL = N - DROWS * NS
CW = 16

R = 1000
NB = N // R


def _sc_mesh():
    return plsc.VectorSubcoreMesh(core_axis_name="c", subcore_axis_name="s")


def _deg_count(dst3, zo):
    @functools.partial(
        pl.kernel,
        out_type=jax.ShapeDtypeStruct((NC, N, CW), jnp.float32),
        mesh=_sc_mesh(),
        scratch_types=[
            pltpu.VMEM((NCH, CHUNK), jnp.int32),
            pltpu.VMEM((2, CHUNK, CW), jnp.float32),
            pltpu.VMEM_SHARED((ACC_ROWS, CW), jnp.float32),
        ],
    )
    def k(dst_hbm, zo_hbm, out_hbm, idx_v, zo_v, acc_s):
        c = lax.axis_index("c")
        s = lax.axis_index("s")
        wid = s * NC + c
        pltpu.sync_copy(zo_hbm, zo_v)
        pltpu.sync_copy(dst_hbm.at[wid], idx_v)
        base = s * ZROWS
        for t in range(ZROWS // CHUNK):
            pltpu.sync_copy(zo_v.at[0], acc_s.at[pl.ds(base + t * CHUNK, CHUNK)])
        rem = ZROWS % CHUNK
        if rem:
            pltpu.sync_copy(zo_v.at[0, pl.ds(0, rem)],
                            acc_s.at[pl.ds(base + ZROWS - rem, rem)])
        plsc.subcore_barrier()

        def body(i, carry):
            pltpu.sync_copy(zo_v.at[1], acc_s.at[idx_v.at[i]], add=True)
            return carry

        lax.fori_loop(0, NCH, body, 0)
        plsc.subcore_barrier()
        pltpu.sync_copy(acc_s.at[pl.ds(s * DROWS, DROWS)],
                        out_hbm.at[c, pl.ds(s * DROWS, DROWS)])

        @pl.when(s == 0)
        def _():
            pltpu.sync_copy(acc_s.at[pl.ds(DROWS * NS, DTAIL)],
                            out_hbm.at[c, pl.ds(DROWS * NS, DTAIL)])

    return k(dst3, zo)


def _edge_pass(p, src3, dst3, zeros_rows):
    @functools.partial(
        pl.kernel,
        out_type=jax.ShapeDtypeStruct((NC, N, H), jnp.float32),
        mesh=_sc_mesh(),
        scratch_types=[
            pltpu.VMEM((HALF, CHUNK), jnp.int32),
            pltpu.VMEM((HALF, CHUNK), jnp.int32),
            pltpu.VMEM((CHUNK, H), jnp.float32),
            pltpu.VMEM((CHUNK, H), jnp.float32),
            pltpu.VMEM_SHARED((ACC_ROWS, H), jnp.float32),
            pltpu.SemaphoreType.DMA,
            pltpu.SemaphoreType.DMA,
        ],
    )
    def k(p_hbm, src_hbm, dst_hbm, z_hbm, out_hbm,
          src_v, dst_v, buf0, buf1, acc_s, sem0, sem1):
        c = lax.axis_index("c")
        s = lax.axis_index("s")
        wid = s * NC + c
        bufs = (buf0, buf1)
        sems = (sem0, sem1)
        pltpu.sync_copy(src_hbm.at[wid, pl.ds(0, HALF)], src_v)
        for b in range(1, NBUF):
            pltpu.async_copy(p_hbm.at[src_v.at[b]], bufs[b], sems[b])
        pltpu.sync_copy(dst_hbm.at[wid, pl.ds(0, HALF)], dst_v)
        pltpu.sync_copy(z_hbm, buf0)
        base = s * ZROWS
        for t in range(ZROWS // CHUNK):
            pltpu.sync_copy(buf0, acc_s.at[pl.ds(base + t * CHUNK, CHUNK)])
        rem = ZROWS % CHUNK
        if rem:
            pltpu.sync_copy(buf0.at[pl.ds(0, rem)],
                            acc_s.at[pl.ds(base + ZROWS - rem, rem)])
        pltpu.async_copy(p_hbm.at[src_v.at[0]], buf0, sem0)
        plsc.subcore_barrier()

        def run_half():
            def body(it, carry):
                g = it * NBUF
                for b in range(NBUF):
                    i = g + b
                    pltpu.make_async_copy(p_hbm.at[src_v.at[i]], bufs[b],
                                          sems[b]).wait()
                    pltpu.sync_copy(bufs[b], acc_s.at[dst_v.at[i]], add=True)
                    nxt = i + NBUF

                    @pl.when(nxt < HALF)
                    def _():
                        pltpu.async_copy(p_hbm.at[src_v.at[nxt]],
                                         bufs[b], sems[b])
                return carry

            lax.fori_loop(0, HALF // NBUF, body, 0)

        run_half()
        pltpu.sync_copy(src_hbm.at[wid, pl.ds(HALF, HALF)], src_v)
        pltpu.sync_copy(dst_hbm.at[wid, pl.ds(HALF, HALF)], dst_v)
        for b in range(NBUF):
            pltpu.async_copy(p_hbm.at[src_v.at[b]], bufs[b], sems[b])
        run_half()
        plsc.subcore_barrier()
        pltpu.sync_copy(acc_s.at[pl.ds(s * DROWS, DROWS)],
                        out_hbm.at[c, pl.ds(s * DROWS, DROWS)])

        @pl.when(s == 0)
        def _():
            pltpu.sync_copy(acc_s.at[pl.ds(DROWS * NS, DTAIL)],
                            out_hbm.at[c, pl.ds(DROWS * NS, DTAIL)])

    return k(p, src3, dst3, zeros_rows)


def _dinv(cnt):
    return lax.rsqrt(cnt[0, :, 0:1] + cnt[1, :, 0:1] + 1.0)


def _prep1_body(cnt_ref, x_ref, w_ref, p_ref):
    d = _dinv(cnt_ref[...])
    p_ref[...] = jnp.dot(x_ref[...], w_ref[...],
                         preferred_element_type=jnp.float32) * d


def _prep1(cnt, x, W1):
    return pl.pallas_call(
        _prep1_body,
        grid=(NB,),
        in_specs=[
            pl.BlockSpec((NC, R, CW), lambda i: (0, i, 0)),
            pl.BlockSpec((R, D), lambda i: (i, 0)),
            pl.BlockSpec((D, H), lambda i: (0, 0)),
        ],
        out_specs=pl.BlockSpec((R, H), lambda i: (i, 0)),
        out_shape=jax.ShapeDtypeStruct((N, H), jnp.float32),
    )(cnt, x, W1)


def _combine_body(cnt_ref, acc_ref, p_ref, b_ref, w_ref, o_ref):
    d = _dinv(cnt_ref[...])
    h = jax.nn.relu(d * (acc_ref[0] + acc_ref[1] + p_ref[...]) + b_ref[...])
    o_ref[...] = jnp.dot(h, w_ref[...], preferred_element_type=jnp.float32) * d


def _combine_prep(cnt, acc, p, b1, W2):
    return pl.pallas_call(
        _combine_body,
        grid=(NB,),
        in_specs=[
            pl.BlockSpec((NC, R, CW), lambda i: (0, i, 0)),
            pl.BlockSpec((NC, R, H), lambda i: (0, i, 0)),
            pl.BlockSpec((R, H), lambda i: (i, 0)),
            pl.BlockSpec((1, H), lambda i: (0, 0)),
            pl.BlockSpec((H, H), lambda i: (0, 0)),
        ],
        out_specs=pl.BlockSpec((R, H), lambda i: (i, 0)),
        out_shape=jax.ShapeDtypeStruct((N, H), jnp.float32),
    )(cnt, acc, p, b1, W2)


def _final_body(cnt_ref, acc_ref, p_ref, b_ref, bat_ref,
                wt_ref, bo_ref, o_ref, pool_acc, cnt_acc):
    i = pl.program_id(0)
    d = _dinv(cnt_ref[...])
    h = jax.nn.relu(d * (acc_ref[0] + acc_ref[1] + p_ref[...]) + b_ref[...])
    bat = bat_ref[0, 0, :]
    oh = (bat[:, None] == lax.broadcasted_iota(jnp.int32, (R, G), 1)
          ).astype(jnp.float32)
    pool = lax.dot_general(oh, h, (((0,), (0,)), ((), ())),
                           preferred_element_type=jnp.float32)
    cnt = jnp.broadcast_to(jnp.sum(oh, axis=0)[:, None], (G, H))

    @pl.when(i == 0)
    def _():
        pool_acc[...] = pool
        cnt_acc[...] = cnt

    @pl.when(i > 0)
    def _():
        pool_acc[...] += pool
        cnt_acc[...] += cnt

    @pl.when(i == NB - 1)
    def _():
        pooled = pool_acc[...] / jnp.maximum(cnt_acc[...], 1.0)
        z = jnp.sum(pooled * wt_ref[...], axis=1, keepdims=True) + bo_ref[...]
        o_ref[...] = 1.0 / (1.0 + jnp.exp(-z))


def _final(cnt, acc, p, b2, bat3, WoutT, bout):
    return pl.pallas_call(
        _final_body,
        grid=(NB,),
        in_specs=[
            pl.BlockSpec((NC, R, CW), lambda i: (0, i, 0)),
            pl.BlockSpec((NC, R, H), lambda i: (0, i, 0)),
            pl.BlockSpec((R, H), lambda i: (i, 0)),
            pl.BlockSpec((1, H), lambda i: (0, 0)),
            pl.BlockSpec((1, 1, R), lambda i: (i, 0, 0)),
            pl.BlockSpec((1, H), lambda i: (0, 0)),
            pl.BlockSpec((1, 1), lambda i: (0, 0)),
        ],
        out_specs=pl.BlockSpec((G, 1), lambda i: (0, 0)),
        out_shape=jax.ShapeDtypeStruct((G, 1), jnp.float32),
        scratch_shapes=[
            pltpu.VMEM((G, H), jnp.float32),
            pltpu.VMEM((G, H), jnp.float32),
        ],
    )(cnt, acc, p, b2, bat3, WoutT, bout)


def kernel(x, edge_index, batch, W1, b1, W2, b2, W_out, b_out):
    pad_ids = jnp.arange(PAD, dtype=jnp.int32)
    src_pad = pad_ids % jnp.int32(N)
    dst_pad = jnp.int32(N) + (pad_ids % jnp.int32(NS))
    src3 = jnp.concatenate([edge_index[0], src_pad]).reshape(NW, NCH, CHUNK)
    dst3 = jnp.concatenate([edge_index[1], dst_pad]).reshape(NW, NCH, CHUNK)

    zo = jnp.stack([jnp.zeros((CHUNK, CW), jnp.float32),
                    jnp.ones((CHUNK, CW), jnp.float32)])
    zeros_rows = jnp.zeros((CHUNK, H), jnp.float32)

    cnt = _deg_count(dst3, zo)

    p1 = _prep1(cnt, x, W1)
    acc1 = _edge_pass(p1, src3, dst3, zeros_rows)
    p2 = _combine_prep(cnt, acc1, p1, b1.reshape(1, H), W2)
    acc2 = _edge_pass(p2, src3, dst3, zeros_rows)

    bat3 = batch.reshape(NB, 1, R)
    out = _final(cnt, acc2, p2, b2.reshape(1, H),
                 bat3, W_out.reshape(1, H), b_out.reshape(1, 1))
    return out


# speedup vs baseline: 1.2998x; 1.0176x over previous
import functools

import jax
import jax.numpy as jnp
from jax import lax
from jax.experimental import pallas as pl
from jax.experimental.pallas import tpu as pltpu
from jax.experimental.pallas import tpu_sc as plsc

N = 10000
D = 128
H = 128
G = 64
E = 320000

NC = 2
NS = 16
NW = NC * NS
CHUNK = 128
NCH = 80
NBUF = 2
HALF = NCH // 2
E_PAD = NW * NCH * CHUNK
PAD = E_PAD - E
ACC_ROWS = 10240
ZROWS = ACC_ROWS // NS
DROWS = 624
DTAIL = N - DROWS * NS
CW = 16

R = 2000
NB = N // R


def _sc_mesh():
    return plsc.VectorSubcoreMesh(core_axis_name="c", subcore_axis_name="s")


def _deg_count(dst3, zo):
    @functools.partial(
        pl.kernel,
        out_type=jax.ShapeDtypeStruct((NC, N, CW), jnp.float32),
        mesh=_sc_mesh(),
        scratch_types=[
            pltpu.VMEM((NCH, CHUNK), jnp.int32),
            pltpu.VMEM((2, CHUNK, CW), jnp.float32),
            pltpu.VMEM_SHARED((ACC_ROWS, CW), jnp.float32),
        ],
    )
    def k(dst_hbm, zo_hbm, out_hbm, idx_v, zo_v, acc_s):
        c = lax.axis_index("c")
        s = lax.axis_index("s")
        wid = s * NC + c
        pltpu.sync_copy(zo_hbm, zo_v)
        pltpu.sync_copy(dst_hbm.at[wid], idx_v)
        base = s * ZROWS
        for t in range(ZROWS // CHUNK):
            pltpu.sync_copy(zo_v.at[0], acc_s.at[pl.ds(base + t * CHUNK, CHUNK)])
        rem = ZROWS % CHUNK
        if rem:
            pltpu.sync_copy(zo_v.at[0, pl.ds(0, rem)],
                            acc_s.at[pl.ds(base + ZROWS - rem, rem)])
        plsc.subcore_barrier()

        def body(i, carry):
            pltpu.sync_copy(zo_v.at[1], acc_s.at[idx_v.at[i]], add=True)
            return carry

        lax.fori_loop(0, NCH, body, 0)
        plsc.subcore_barrier()
        pltpu.sync_copy(acc_s.at[pl.ds(s * DROWS, DROWS)],
                        out_hbm.at[c, pl.ds(s * DROWS, DROWS)])

        @pl.when(s == 0)
        def _():
            pltpu.sync_copy(acc_s.at[pl.ds(DROWS * NS, DTAIL)],
                            out_hbm.at[c, pl.ds(DROWS * NS, DTAIL)])

    return k(dst3, zo)


def _edge_pass(p, src3, dst3, zeros_rows):
    @functools.partial(
        pl.kernel,
        out_type=jax.ShapeDtypeStruct((NC, N, H), jnp.float32),
        mesh=_sc_mesh(),
        scratch_types=[
            pltpu.VMEM((HALF, CHUNK), jnp.int32),
            pltpu.VMEM((HALF, CHUNK), jnp.int32),
            pltpu.VMEM((CHUNK, H), jnp.float32),
            pltpu.VMEM((CHUNK, H), jnp.float32),
            pltpu.VMEM_SHARED((ACC_ROWS, H), jnp.float32),
            pltpu.SemaphoreType.DMA,
            pltpu.SemaphoreType.DMA,
        ],
    )
    def k(p_hbm, src_hbm, dst_hbm, z_hbm, out_hbm,
          src_v, dst_v, buf0, buf1, acc_s, sem0, sem1):
        c = lax.axis_index("c")
        s = lax.axis_index("s")
        wid = s * NC + c
        bufs = (buf0, buf1)
        sems = (sem0, sem1)
        pltpu.sync_copy(src_hbm.at[wid, pl.ds(0, HALF)], src_v)
        for b in range(1, NBUF):
            pltpu.async_copy(p_hbm.at[src_v.at[b]], bufs[b], sems[b])
        pltpu.sync_copy(dst_hbm.at[wid, pl.ds(0, HALF)], dst_v)
        pltpu.sync_copy(z_hbm, buf0)
        base = s * ZROWS
        for t in range(ZROWS // CHUNK):
            pltpu.sync_copy(buf0, acc_s.at[pl.ds(base + t * CHUNK, CHUNK)])
        rem = ZROWS % CHUNK
        if rem:
            pltpu.sync_copy(buf0.at[pl.ds(0, rem)],
                            acc_s.at[pl.ds(base + ZROWS - rem, rem)])
        pltpu.async_copy(p_hbm.at[src_v.at[0]], buf0, sem0)
        plsc.subcore_barrier()

        def run_half():
            def body(it, carry):
                g = it * NBUF
                for b in range(NBUF):
                    i = g + b
                    pltpu.make_async_copy(p_hbm.at[src_v.at[i]], bufs[b],
                                          sems[b]).wait()
                    pltpu.sync_copy(bufs[b], acc_s.at[dst_v.at[i]], add=True)
                    nxt = i + NBUF

                    @pl.when(nxt < HALF)
                    def _():
                        pltpu.async_copy(p_hbm.at[src_v.at[nxt]],
                                         bufs[b], sems[b])
                return carry

            lax.fori_loop(0, HALF // NBUF, body, 0)

        run_half()
        pltpu.sync_copy(src_hbm.at[wid, pl.ds(HALF, HALF)], src_v)
        pltpu.sync_copy(dst_hbm.at[wid, pl.ds(HALF, HALF)], dst_v)
        for b in range(NBUF):
            pltpu.async_copy(p_hbm.at[src_v.at[b]], bufs[b], sems[b])
        run_half()
        plsc.subcore_barrier()
        pltpu.sync_copy(acc_s.at[pl.ds(s * DROWS, DROWS)],
                        out_hbm.at[c, pl.ds(s * DROWS, DROWS)])

        @pl.when(s == 0)
        def _():
            pltpu.sync_copy(acc_s.at[pl.ds(DROWS * NS, DTAIL)],
                            out_hbm.at[c, pl.ds(DROWS * NS, DTAIL)])

    return k(p, src3, dst3, zeros_rows)


def _dinv(cnt):
    return lax.rsqrt(cnt[0, :, 0:1] + cnt[1, :, 0:1] + 1.0)


def _prep1_body(cnt_ref, x_ref, w_ref, p_ref):
    d = _dinv(cnt_ref[...])
    p_ref[...] = jnp.dot(x_ref[...], w_ref[...],
                         preferred_element_type=jnp.float32) * d


def _prep1(cnt, x, W1):
    return pl.pallas_call(
        _prep1_body,
        grid=(NB,),
        in_specs=[
            pl.BlockSpec((NC, R, CW), lambda i: (0, i, 0)),
            pl.BlockSpec((R, D), lambda i: (i, 0)),
            pl.BlockSpec((D, H), lambda i: (0, 0)),
        ],
        out_specs=pl.BlockSpec((R, H), lambda i: (i, 0)),
        out_shape=jax.ShapeDtypeStruct((N, H), jnp.float32),
    )(cnt, x, W1)


def _combine_body(cnt_ref, acc_ref, p_ref, b_ref, w_ref, o_ref):
    d = _dinv(cnt_ref[...])
    h = jax.nn.relu(d * (acc_ref[0] + acc_ref[1] + p_ref[...]) + b_ref[...])
    o_ref[...] = jnp.dot(h, w_ref[...], preferred_element_type=jnp.float32) * d


def _combine_prep(cnt, acc, p, b1, W2):
    return pl.pallas_call(
        _combine_body,
        grid=(NB,),
        in_specs=[
            pl.BlockSpec((NC, R, CW), lambda i: (0, i, 0)),
            pl.BlockSpec((NC, R, H), lambda i: (0, i, 0)),
            pl.BlockSpec((R, H), lambda i: (i, 0)),
            pl.BlockSpec((1, H), lambda i: (0, 0)),
            pl.BlockSpec((H, H), lambda i: (0, 0)),
        ],
        out_specs=pl.BlockSpec((R, H), lambda i: (i, 0)),
        out_shape=jax.ShapeDtypeStruct((N, H), jnp.float32),
    )(cnt, acc, p, b1, W2)


def _final_body(cnt_ref, acc_ref, p_ref, b_ref, bat_ref,
                wt_ref, bo_ref, o_ref, pool_acc, cnt_acc):
    i = pl.program_id(0)
    d = _dinv(cnt_ref[...])
    h = jax.nn.relu(d * (acc_ref[0] + acc_ref[1] + p_ref[...]) + b_ref[...])
    bat = bat_ref[0, 0, :]
    oh = (bat[:, None] == lax.broadcasted_iota(jnp.int32, (R, G), 1)
          ).astype(jnp.float32)
    pool = lax.dot_general(oh, h, (((0,), (0,)), ((), ())),
                           preferred_element_type=jnp.float32)
    cnt = jnp.broadcast_to(jnp.sum(oh, axis=0)[:, None], (G, H))

    @pl.when(i == 0)
    def _():
        pool_acc[...] = pool
        cnt_acc[...] = cnt

    @pl.when(i > 0)
    def _():
        pool_acc[...] += pool
        cnt_acc[...] += cnt

    @pl.when(i == NB - 1)
    def _():
        pooled = pool_acc[...] / jnp.maximum(cnt_acc[...], 1.0)
        z = jnp.sum(pooled * wt_ref[...], axis=1, keepdims=True) + bo_ref[...]
        o_ref[...] = 1.0 / (1.0 + jnp.exp(-z))


def _final(cnt, acc, p, b2, bat3, WoutT, bout):
    return pl.pallas_call(
        _final_body,
        grid=(NB,),
        in_specs=[
            pl.BlockSpec((NC, R, CW), lambda i: (0, i, 0)),
            pl.BlockSpec((NC, R, H), lambda i: (0, i, 0)),
            pl.BlockSpec((R, H), lambda i: (i, 0)),
            pl.BlockSpec((1, H), lambda i: (0, 0)),
            pl.BlockSpec((1, 1, R), lambda i: (i, 0, 0)),
            pl.BlockSpec((1, H), lambda i: (0, 0)),
            pl.BlockSpec((1, 1), lambda i: (0, 0)),
        ],
        out_specs=pl.BlockSpec((G, 1), lambda i: (0, 0)),
        out_shape=jax.ShapeDtypeStruct((G, 1), jnp.float32),
        scratch_shapes=[
            pltpu.VMEM((G, H), jnp.float32),
            pltpu.VMEM((G, H), jnp.float32),
        ],
    )(cnt, acc, p, b2, bat3, WoutT, bout)


def kernel(x, edge_index, batch, W1, b1, W2, b2, W_out, b_out):
    pad_ids = jnp.arange(PAD, dtype=jnp.int32)
    src_pad = pad_ids % jnp.int32(N)
    dst_pad = jnp.int32(N) + (pad_ids % jnp.int32(NS))
    src3 = jnp.concatenate([edge_index[0], src_pad]).reshape(NW, NCH, CHUNK)
    dst3 = jnp.concatenate([edge_index[1], dst_pad]).reshape(NW, NCH, CHUNK)

    zo = jnp.stack([jnp.zeros((CHUNK, CW), jnp.float32),
                    jnp.ones((CHUNK, CW), jnp.float32)])
    zeros_rows = jnp.zeros((CHUNK, H), jnp.float32)

    cnt = _deg_count(dst3, zo)

    p1 = _prep1(cnt, x, W1)
    acc1 = _edge_pass(p1, src3, dst3, zeros_rows)
    p2 = _combine_prep(cnt, acc1, p1, b1.reshape(1, H), W2)
    acc2 = _edge_pass(p2, src3, dst3, zeros_rows)

    bat3 = batch.reshape(NB, 1, R)
    out = _final(cnt, acc2, p2, b2.reshape(1, H),
                 bat3, W_out.reshape(1, H), b_out.reshape(1, 1))
    return out
